# baseline (speedup 1.0000x reference)
import functools

import jax
import jax.numpy as jnp
from jax import lax
from jax.experimental import pallas as pl
from jax.experimental.pallas import tpu as pltpu
from jax.experimental.pallas import tpu_sc as plsc

_B, _S, _D = 8, 1024, 384
_K = 1024
_N = _B * _S
_TS = 256
_CHUNK_BLOCKS = (8, 8, 8, 8)


def _tc_body(x_ref, ct_ref, idx_ref, loss_ref):
    i = pl.program_id(0)
    xb = x_ref[...]
    ct = ct_ref[...]
    xc = lax.dot_general(
        xb, ct, (((1,), (0,)), ((), ())),
        preferred_element_type=jnp.float32,
    )
    x2 = jnp.sum(xb * xb, axis=1, keepdims=True)
    c2 = jnp.sum(ct * ct, axis=0, keepdims=True)
    dist = x2 + c2 - 2.0 * xc
    dmin = jnp.min(dist, axis=1, keepdims=True)
    iota = lax.broadcasted_iota(jnp.int32, (_TS, _K), 1) | jnp.int32(0x3F800000)
    masked = jnp.where(dist == dmin, iota, jnp.int32(0x3F800000 | _K))
    keymin = jnp.min(lax.bitcast_convert_type(masked, jnp.float32), axis=1)
    idx = lax.bitcast_convert_type(keymin, jnp.int32) & jnp.int32(0xFFFF)
    idx_ref[0, 0, :] = idx

    @pl.when(i == 0)
    def _():
        loss_ref[...] = jnp.zeros((1, 1), jnp.float32)

    loss_ref[...] += jnp.sum(dmin, keepdims=True)


@functools.cache
def _make_tc_call(block_base, nblocks):
    return pl.pallas_call(
        _tc_body,
        grid=(nblocks,),
        in_specs=[
            pl.BlockSpec((_TS, _D), lambda i: (block_base + i, 0)),
            pl.BlockSpec((_D, _K), lambda i: (0, 0)),
        ],
        out_specs=[
            pl.BlockSpec((1, 1, _TS), lambda i: (i, 0, 0)),
            pl.BlockSpec((1, 1), lambda i: (0, 0)),
        ],
        out_shape=[
            jax.ShapeDtypeStruct((nblocks, 1, _TS), jnp.int32),
            jax.ShapeDtypeStruct((1, 1), jnp.float32),
        ],
    )


@functools.cache
def _make_sc_gather(chunk_base, ntok):
    info = plsc.get_sparse_core_info()
    nc, ns = info.num_cores, info.num_subcores
    nw = nc * ns
    bpw = ntok // nw
    mesh = plsc.VectorSubcoreMesh(core_axis_name="c", subcore_axis_name="s")

    @functools.partial(
        pl.kernel,
        mesh=mesh,
        out_type=(),
        scratch_types=[
            pltpu.VMEM((bpw,), jnp.int32),
            pltpu.VMEM((bpw, _D), jnp.float32),
            pltpu.SemaphoreType.DMA,
        ],
    )
    def gather(idx_hbm, table_hbm, out_ref, idx_v, rows_v, sem):
        wid = lax.axis_index("s") * nc + lax.axis_index("c")
        base = wid * bpw
        pltpu.sync_copy(idx_hbm.at[pl.ds(base, bpw)], idx_v)
        pltpu.async_copy(table_hbm.at[idx_v], rows_v, sem).wait()
        pltpu.sync_copy(rows_v, out_ref.at[pl.ds(chunk_base + base, bpw)])

    return gather


def kernel(x, codebook):
    xf = x.reshape(_N, _D)
    ct = codebook.T
    out_ref = jax.empty_ref(jax.ShapeDtypeStruct((_N, _D), jnp.float32))
    idx_chunks = []
    loss = None
    block_base = 0
    for nb in _CHUNK_BLOCKS:
        ntok = nb * _TS
        idx3, lsum = _make_tc_call(block_base, nb)(xf, ct)
        idx_flat = idx3.reshape(ntok)
        idx_chunks.append(idx_flat)
        _make_sc_gather(block_base * _TS, ntok)(idx_flat, codebook, out_ref)
        loss = lsum if loss is None else loss + lsum
        block_base += nb
    indices = jnp.concatenate(idx_chunks).reshape(_B, _S)
    x_quantized = jax.freeze(out_ref).reshape(_B, _S, _D)
    commit_loss = loss[0, 0] / jnp.float32(_N * _D)
    return (x_quantized, indices, commit_loss)


# speedup vs baseline: 1.1108x; 1.1108x over previous
import functools

import jax
import jax.numpy as jnp
from jax import lax
from jax.experimental import pallas as pl
from jax.experimental.pallas import tpu as pltpu
from jax.experimental.pallas import tpu_sc as plsc

_B, _S, _D = 8, 1024, 384
_K = 1024
_N = _B * _S
_TS = 512
_CHUNK_BLOCKS = (4, 4, 4, 4)


def _tc_body(x_ref, ct_ref, idx_ref, loss_ref):
    i = pl.program_id(0)
    xb = x_ref[...]
    ct = ct_ref[...]
    xc = lax.dot_general(
        xb, ct, (((1,), (0,)), ((), ())),
        preferred_element_type=jnp.float32,
    )
    x2 = jnp.sum(xb * xb, axis=1, keepdims=True)
    c2 = jnp.sum(ct * ct, axis=0, keepdims=True)
    dist = x2 + c2 - 2.0 * xc
    dmin = jnp.min(dist, axis=1, keepdims=True)
    iota = lax.broadcasted_iota(jnp.int32, (_TS, _K), 1) | jnp.int32(0x3F800000)
    masked = jnp.where(dist == dmin, iota, jnp.int32(0x3F800000 | _K))
    keymin = jnp.min(lax.bitcast_convert_type(masked, jnp.float32), axis=1)
    idx = lax.bitcast_convert_type(keymin, jnp.int32) & jnp.int32(0xFFFF)
    idx_ref[0, 0, :] = idx

    @pl.when(i == 0)
    def _():
        loss_ref[...] = jnp.zeros((1, 1), jnp.float32)

    loss_ref[...] += jnp.sum(dmin, keepdims=True)


@functools.cache
def _make_tc_call(block_base, nblocks):
    return pl.pallas_call(
        _tc_body,
        grid=(nblocks,),
        in_specs=[
            pl.BlockSpec((_TS, _D), lambda i: (block_base + i, 0)),
            pl.BlockSpec((_D, _K), lambda i: (0, 0)),
        ],
        out_specs=[
            pl.BlockSpec((1, 1, _TS), lambda i: (i, 0, 0)),
            pl.BlockSpec((1, 1), lambda i: (0, 0)),
        ],
        out_shape=[
            jax.ShapeDtypeStruct((nblocks, 1, _TS), jnp.int32),
            jax.ShapeDtypeStruct((1, 1), jnp.float32),
        ],
    )


@functools.cache
def _make_sc_gather(chunk_base, ntok):
    info = plsc.get_sparse_core_info()
    nc, ns = info.num_cores, info.num_subcores
    nw = nc * ns
    bpw = ntok // nw
    mesh = plsc.VectorSubcoreMesh(core_axis_name="c", subcore_axis_name="s")

    @functools.partial(
        pl.kernel,
        mesh=mesh,
        out_type=(),
        scratch_types=[
            pltpu.VMEM((bpw,), jnp.int32),
            pltpu.VMEM((bpw, _D), jnp.float32),
            pltpu.SemaphoreType.DMA,
        ],
    )
    def gather(idx_hbm, table_hbm, out_ref, idx_v, rows_v, sem):
        wid = lax.axis_index("s") * nc + lax.axis_index("c")
        base = wid * bpw
        pltpu.sync_copy(idx_hbm.at[pl.ds(base, bpw)], idx_v)
        pltpu.async_copy(table_hbm.at[idx_v], rows_v, sem).wait()
        pltpu.sync_copy(rows_v, out_ref.at[pl.ds(chunk_base + base, bpw)])

    return gather


def kernel(x, codebook):
    xf = x.reshape(_N, _D)
    ct = codebook.T
    out_ref = jax.empty_ref(jax.ShapeDtypeStruct((_N, _D), jnp.float32))
    idx_chunks = []
    loss = None
    block_base = 0
    for nb in _CHUNK_BLOCKS:
        ntok = nb * _TS
        idx3, lsum = _make_tc_call(block_base, nb)(xf, ct)
        idx_flat = idx3.reshape(ntok)
        idx_chunks.append(idx_flat)
        _make_sc_gather(block_base * _TS, ntok)(idx_flat, codebook, out_ref)
        loss = lsum if loss is None else loss + lsum
        block_base += nb
    indices = jnp.concatenate(idx_chunks).reshape(_B, _S)
    x_quantized = jax.freeze(out_ref).reshape(_B, _S, _D)
    commit_loss = loss[0, 0] / jnp.float32(_N * _D)
    return (x_quantized, indices, commit_loss)
